# baseline (speedup 1.0000x reference)
import functools
import math

import jax
import jax.numpy as jnp
from jax import lax
from jax.experimental import pallas as pl
from jax.experimental.pallas import tpu as pltpu
from jax.experimental.pallas import tpu_sc as plsc

_B = 1024
_D = 64
_SCALE = 1.0 / math.sqrt(_D)
_N_BLK = 4096


def _make_sc_gather(V, D, B):
    info = plsc.get_sparse_core_info()
    NC, NS = info.num_cores, info.num_subcores
    NW = NC * NS
    assert B % (8 * NW) == 0 and D % info.num_lanes == 0
    b_per_w = B // NW
    mesh = plsc.VectorSubcoreMesh(core_axis_name="c", subcore_axis_name="s")

    @functools.partial(
        pl.kernel,
        mesh=mesh,
        out_type=jax.ShapeDtypeStruct((B, D), jnp.float32),
        compiler_params=pltpu.CompilerParams(use_tc_tiling_on_sc=False),
        scratch_types=[
            pltpu.VMEM((b_per_w,), jnp.int32),
            pltpu.VMEM((b_per_w, D), jnp.float32),
            pltpu.SemaphoreType.DMA,
        ],
    )
    def gather_k(table_hbm, idx_hbm, out_hbm, idx_v, rows_v, sem):
        wid = lax.axis_index("s") * NC + lax.axis_index("c")
        base = wid * b_per_w
        pltpu.sync_copy(idx_hbm.at[pl.ds(base, b_per_w)], idx_v)
        pltpu.async_copy(table_hbm.at[idx_v], rows_v, sem).wait()
        pltpu.sync_copy(rows_v, out_hbm.at[pl.ds(base, b_per_w)])

    return gather_k


def _mm_body(pe_ref, il_ref, out_ref):
    out_ref[...] = lax.dot_general(
        pe_ref[...], il_ref[...],
        dimension_numbers=(((1,), (1,)), ((), ())),
        preferred_element_type=jnp.float32,
    ) * _SCALE


def _tc_matmul(prev_emb, IL):
    B, D = prev_emb.shape
    V = IL.shape[0]
    grid = (pl.cdiv(V, _N_BLK),)
    return pl.pallas_call(
        _mm_body,
        grid=grid,
        in_specs=[
            pl.BlockSpec((B, D), lambda j: (0, 0)),
            pl.BlockSpec((_N_BLK, D), lambda j: (j, 0)),
        ],
        out_specs=pl.BlockSpec((B, _N_BLK), lambda j: (0, j)),
        out_shape=jax.ShapeDtypeStruct((B, V), jnp.float32),
    )(prev_emb, IL)


def _zero_body(out_ref):
    out_ref[...] = jnp.zeros_like(out_ref)


def kernel(X, tag, IL, LI):
    V = IL.shape[0]
    B_BLK = 32
    return pl.pallas_call(
        _zero_body,
        grid=(_B // B_BLK,),
        out_specs=pl.BlockSpec((B_BLK, V), lambda i: (i, 0)),
        out_shape=jax.ShapeDtypeStruct((_B, V), jnp.float32),
    )()


# speedup vs baseline: 2.1920x; 2.1920x over previous
import functools
import math

import jax
import jax.numpy as jnp
from jax import lax
from jax.experimental import pallas as pl
from jax.experimental.pallas import tpu as pltpu
from jax.experimental.pallas import tpu_sc as plsc

_B = 1024
_D = 64
_DP = 128
_SCALE = 1.0 / math.sqrt(_D)
_V_BLK = 2048


def _make_sc_gather(Vh, B):
    info = plsc.get_sparse_core_info()
    NC, NS = info.num_cores, info.num_subcores
    NW = NC * NS
    assert B % (8 * NW) == 0
    b_per_w = B // NW
    mesh = plsc.VectorSubcoreMesh(core_axis_name="c", subcore_axis_name="s")

    @functools.partial(
        pl.kernel,
        mesh=mesh,
        out_type=jax.ShapeDtypeStruct((B, _DP), jnp.float32),
        scratch_types=[
            pltpu.VMEM((b_per_w,), jnp.int32),
            pltpu.VMEM((b_per_w, _DP), jnp.float32),
            pltpu.SemaphoreType.DMA,
        ],
    )
    def gather_k(table_hbm, idx_hbm, out_hbm, idx_v, rows_v, sem):
        wid = lax.axis_index("s") * NC + lax.axis_index("c")
        base = wid * b_per_w
        pltpu.sync_copy(idx_hbm.at[pl.ds(base, b_per_w)], idx_v)
        pltpu.async_copy(table_hbm.at[idx_v], rows_v, sem).wait()
        pltpu.sync_copy(rows_v, out_hbm.at[pl.ds(base, b_per_w)])

    return gather_k


def _mm_body(il_t_ref, pe_ref, par_ref, out_ref):
    pe128 = pe_ref[...]
    par = par_ref[...] != 0
    pe = jnp.where(par, pe128[:, _D:], pe128[:, :_D])
    out_ref[...] = lax.dot_general(
        il_t_ref[...], pe,
        dimension_numbers=(((0,), (1,)), ((), ())),
        preferred_element_type=jnp.float32,
    ) * _SCALE


def _tc_matmul_t(IL_T, pe128, parity):
    V = IL_T.shape[1]
    return pl.pallas_call(
        _mm_body,
        grid=(pl.cdiv(V, _V_BLK),),
        in_specs=[
            pl.BlockSpec((_D, _V_BLK), lambda j: (0, j)),
            pl.BlockSpec((_B, _DP), lambda j: (0, 0)),
            pl.BlockSpec((_B, 1), lambda j: (0, 0)),
        ],
        out_specs=pl.BlockSpec((_V_BLK, _B), lambda j: (j, 0)),
        out_shape=jax.ShapeDtypeStruct((V, _B), jnp.float32),
    )(IL_T, pe128, parity)


def kernel(X, tag, IL, LI):
    V = IL.shape[0]
    prev_iid = X[:, -2, 3].astype(jnp.int32)
    table = LI.reshape(V // 2, _DP)
    pe128 = _make_sc_gather(V // 2, _B)(table, prev_iid >> 1)
    parity = (prev_iid & 1).reshape(_B, 1)
    out_t = _tc_matmul_t(IL.T, pe128, parity)
    return out_t.T


# speedup vs baseline: 2.2069x; 1.0068x over previous
import functools
import math

import jax
import jax.numpy as jnp
from jax import lax
from jax.experimental import pallas as pl
from jax.experimental.pallas import tpu as pltpu
from jax.experimental.pallas import tpu_sc as plsc

_B = 1024
_D = 64
_DP = 128
_SCALE = 1.0 / math.sqrt(_D)
_V_BLK = 4096


def _make_sc_gather(Vh, B):
    info = plsc.get_sparse_core_info()
    NC, NS = info.num_cores, info.num_subcores
    NW = NC * NS
    assert B % (8 * NW) == 0
    b_per_w = B // NW
    mesh = plsc.VectorSubcoreMesh(core_axis_name="c", subcore_axis_name="s")

    @functools.partial(
        pl.kernel,
        mesh=mesh,
        out_type=jax.ShapeDtypeStruct((B, _DP), jnp.float32),
        scratch_types=[
            pltpu.VMEM((b_per_w,), jnp.int32),
            pltpu.VMEM((b_per_w, _DP), jnp.float32),
            pltpu.SemaphoreType.DMA,
        ],
    )
    def gather_k(table_hbm, idx_hbm, out_hbm, idx_v, rows_v, sem):
        wid = lax.axis_index("s") * NC + lax.axis_index("c")
        base = wid * b_per_w
        pltpu.sync_copy(idx_hbm.at[pl.ds(base, b_per_w)], idx_v)
        pltpu.async_copy(table_hbm.at[idx_v], rows_v, sem).wait()
        pltpu.sync_copy(rows_v, out_hbm.at[pl.ds(base, b_per_w)])

    return gather_k


def _mm_body(il_t_ref, pe_ref, par_ref, out_ref):
    pe128 = pe_ref[...]
    par = par_ref[...] != 0
    pe = jnp.where(par, pe128[:, _D:], pe128[:, :_D])
    out_ref[...] = lax.dot_general(
        il_t_ref[...], pe,
        dimension_numbers=(((0,), (1,)), ((), ())),
        preferred_element_type=jnp.float32,
    ) * _SCALE


def _tc_matmul_t(IL_T, pe128, parity):
    V = IL_T.shape[1]
    return pl.pallas_call(
        _mm_body,
        grid=(pl.cdiv(V, _V_BLK),),
        in_specs=[
            pl.BlockSpec((_D, _V_BLK), lambda j: (0, j)),
            pl.BlockSpec((_B, _DP), lambda j: (0, 0)),
            pl.BlockSpec((_B, 1), lambda j: (0, 0)),
        ],
        out_specs=pl.BlockSpec((_V_BLK, _B), lambda j: (j, 0)),
        out_shape=jax.ShapeDtypeStruct((V, _B), jnp.float32),
    )(IL_T, pe128, parity)


def kernel(X, tag, IL, LI):
    V = IL.shape[0]
    prev_iid = X[:, -2, 3].astype(jnp.int32)
    table = LI.reshape(V // 2, _DP)
    pe128 = _make_sc_gather(V // 2, _B)(table, prev_iid >> 1)
    parity = (prev_iid & 1).reshape(_B, 1)
    out_t = _tc_matmul_t(IL.T, pe128, parity)
    return out_t.T


# speedup vs baseline: 2.7212x; 1.2330x over previous
import functools
import math

import jax
import jax.numpy as jnp
from jax import lax
from jax.experimental import pallas as pl
from jax.experimental.pallas import tpu as pltpu
from jax.experimental.pallas import tpu_sc as plsc

_B = 1024
_D = 64
_SCALE = 1.0 / math.sqrt(_D)
_V_BLK = 4096
_SUB = 12544
_NQ = 4
_QOWN = 24960


def _make_sc_gather_t(V, B):
    info = plsc.get_sparse_core_info()
    NC, NS, L = info.num_cores, info.num_subcores, info.num_lanes
    NW = NC * NS
    assert NW == 32 and _D == 64 and B % L == 0
    main = V - V % 128
    tail = V - main
    assert _NQ * _QOWN + (2 * _SUB - _QOWN) == main
    mesh = plsc.VectorSubcoreMesh(core_axis_name="c", subcore_axis_name="s")

    @functools.partial(
        pl.kernel,
        mesh=mesh,
        out_type=jax.ShapeDtypeStruct((_NQ * _D, B), jnp.float32),
        compiler_params=pltpu.CompilerParams(needs_layout_passes=False),
        scratch_types=[
            pltpu.VMEM((8, _SUB), jnp.float32),
            pltpu.VMEM((tail, _D), jnp.float32),
            pltpu.VMEM((B,), jnp.int32),
            pltpu.VMEM((8, B), jnp.float32),
        ],
    )
    def gather_k(lit_hbm, tail_hbm, idx_hbm, out_hbm, buf_v, tail_v, idx_v,
                 acc_v):
        wid = lax.axis_index("s") * NC + lax.axis_index("c")
        gq = wid // _NQ
        q = wid % _NQ
        row0 = pl.multiple_of(8 * gq, 8)
        own_lo = q * _QOWN
        own_hi = jnp.where(q == _NQ - 1, own_lo + 2 * _SUB, own_lo + _QOWN)
        pltpu.sync_copy(idx_hbm, idx_v)
        pltpu.sync_copy(tail_hbm, tail_v)
        for k in range(2):
            lo = pl.multiple_of(own_lo + k * _SUB, 128)
            pltpu.sync_copy(lit_hbm.at[pl.ds(row0, 8), pl.ds(lo, _SUB)],
                            buf_v)

            def g_body(g, _, k=k, lo=lo):
                iv = idx_v[pl.ds(g * L, L)]
                m = (iv >= lo) & (iv < lo + _SUB) & (iv < own_hi)
                ig = jnp.where(m, iv - lo, 0)
                for rr in range(8):
                    rv = jnp.full((L,), rr, jnp.int32)
                    x = plsc.load_gather(buf_v, [rv, ig], mask=m)
                    prev = (jnp.zeros_like(x) if k == 0
                            else acc_v[rr, pl.ds(g * L, L)])
                    acc_v[rr, pl.ds(g * L, L)] = jnp.where(m, x, prev)
                return 0

            lax.fori_loop(0, B // L, g_body, 0, unroll=False)

        def t_body(g, _):
            iv = idx_v[pl.ds(g * L, L)]
            m = (iv >= main) & (q == _NQ - 1)
            ig = jnp.where(m, iv - main, 0)
            for rr in range(8):
                rv = jnp.full((L,), rr, jnp.int32)
                x = plsc.load_gather(tail_v, [ig, row0 + rv], mask=m)
                prev = acc_v[rr, pl.ds(g * L, L)]
                acc_v[rr, pl.ds(g * L, L)] = jnp.where(m, x, prev)
            return 0

        lax.fori_loop(0, B // L, t_body, 0, unroll=False)
        pltpu.sync_copy(acc_v, out_hbm.at[pl.ds(pl.multiple_of(q * _D + row0, 8), 8)])

    return gather_k


def _mm_body(il_t_ref, pe4_ref, out_ref):
    pe = (pe4_ref[pl.ds(0, _D), :] + pe4_ref[pl.ds(_D, _D), :]
          + pe4_ref[pl.ds(2 * _D, _D), :] + pe4_ref[pl.ds(3 * _D, _D), :])
    out_ref[...] = lax.dot_general(
        il_t_ref[...], pe,
        dimension_numbers=(((0,), (0,)), ((), ())),
        preferred_element_type=jnp.float32,
    ) * _SCALE


def _tc_matmul_t(IL_T, pe4):
    V = IL_T.shape[1]
    return pl.pallas_call(
        _mm_body,
        grid=(pl.cdiv(V, _V_BLK),),
        in_specs=[
            pl.BlockSpec((_D, _V_BLK), lambda j: (0, j)),
            pl.BlockSpec((_NQ * _D, _B), lambda j: (0, 0)),
        ],
        out_specs=pl.BlockSpec((_V_BLK, _B), lambda j: (j, 0)),
        out_shape=jax.ShapeDtypeStruct((V, _B), jnp.float32),
        compiler_params=pltpu.CompilerParams(
            vmem_limit_bytes=56 * 1024 * 1024),
    )(IL_T, pe4)


def kernel(X, tag, IL, LI):
    V = IL.shape[0]
    main = V - V % 128
    prev_iid = X[:, -2, 3].astype(jnp.int32)
    tail = LI[main:, :]
    pe4 = _make_sc_gather_t(V, _B)(LI.T, tail, prev_iid)
    out_t = _tc_matmul_t(IL.T, pe4)
    return out_t.T


# speedup vs baseline: 2.7266x; 1.0020x over previous
import functools
import math

import jax
import jax.numpy as jnp
from jax import lax
from jax.experimental import pallas as pl
from jax.experimental.pallas import tpu as pltpu
from jax.experimental.pallas import tpu_sc as plsc

_B = 1024
_D = 64
_SCALE = 1.0 / math.sqrt(_D)
_V_BLK = 6144
_SUB = 12544
_NQ = 4
_QOWN = 24960


def _make_sc_gather_t(V, B):
    info = plsc.get_sparse_core_info()
    NC, NS, L = info.num_cores, info.num_subcores, info.num_lanes
    NW = NC * NS
    assert NW == 32 and _D == 64 and B % L == 0
    main = V - V % 128
    tail = V - main
    assert _NQ * _QOWN + (2 * _SUB - _QOWN) == main
    mesh = plsc.VectorSubcoreMesh(core_axis_name="c", subcore_axis_name="s")

    @functools.partial(
        pl.kernel,
        mesh=mesh,
        out_type=jax.ShapeDtypeStruct((_NQ * _D, B), jnp.float32),
        compiler_params=pltpu.CompilerParams(needs_layout_passes=False),
        scratch_types=[
            pltpu.VMEM((8, _SUB), jnp.float32),
            pltpu.VMEM((tail, _D), jnp.float32),
            pltpu.VMEM((B,), jnp.int32),
            pltpu.VMEM((8, B), jnp.float32),
        ],
    )
    def gather_k(lit_hbm, tail_hbm, idx_hbm, out_hbm, buf_v, tail_v, idx_v,
                 acc_v):
        wid = lax.axis_index("s") * NC + lax.axis_index("c")
        gq = wid // _NQ
        q = wid % _NQ
        row0 = pl.multiple_of(8 * gq, 8)
        own_lo = q * _QOWN
        own_hi = jnp.where(q == _NQ - 1, own_lo + 2 * _SUB, own_lo + _QOWN)
        pltpu.sync_copy(idx_hbm, idx_v)
        pltpu.sync_copy(tail_hbm, tail_v)
        for k in range(2):
            lo = pl.multiple_of(own_lo + k * _SUB, 128)
            pltpu.sync_copy(lit_hbm.at[pl.ds(row0, 8), pl.ds(lo, _SUB)],
                            buf_v)

            def g_body(g, _, k=k, lo=lo):
                iv = idx_v[pl.ds(g * L, L)]
                m = (iv >= lo) & (iv < lo + _SUB) & (iv < own_hi)
                ig = jnp.where(m, iv - lo, 0)
                for rr in range(8):
                    rv = jnp.full((L,), rr, jnp.int32)
                    x = plsc.load_gather(buf_v, [rv, ig], mask=m)
                    prev = (jnp.zeros_like(x) if k == 0
                            else acc_v[rr, pl.ds(g * L, L)])
                    acc_v[rr, pl.ds(g * L, L)] = jnp.where(m, x, prev)
                return 0

            lax.fori_loop(0, B // L, g_body, 0, unroll=False)

        def t_body(g, _):
            iv = idx_v[pl.ds(g * L, L)]
            m = (iv >= main) & (q == _NQ - 1)
            ig = jnp.where(m, iv - main, 0)
            for rr in range(8):
                rv = jnp.full((L,), rr, jnp.int32)
                x = plsc.load_gather(tail_v, [ig, row0 + rv], mask=m)
                prev = acc_v[rr, pl.ds(g * L, L)]
                acc_v[rr, pl.ds(g * L, L)] = jnp.where(m, x, prev)
            return 0

        lax.fori_loop(0, B // L, t_body, 0, unroll=False)
        pltpu.sync_copy(acc_v, out_hbm.at[pl.ds(pl.multiple_of(q * _D + row0, 8), 8)])

    return gather_k


def _mm_body(il_t_ref, pe4_ref, out_ref):
    pe = (pe4_ref[pl.ds(0, _D), :] + pe4_ref[pl.ds(_D, _D), :]
          + pe4_ref[pl.ds(2 * _D, _D), :] + pe4_ref[pl.ds(3 * _D, _D), :])
    out_ref[...] = lax.dot_general(
        il_t_ref[...], pe,
        dimension_numbers=(((0,), (0,)), ((), ())),
        preferred_element_type=jnp.float32,
    ) * _SCALE


def _tc_matmul_t(IL_T, pe4):
    V = IL_T.shape[1]
    return pl.pallas_call(
        _mm_body,
        grid=(pl.cdiv(V, _V_BLK),),
        in_specs=[
            pl.BlockSpec((_D, _V_BLK), lambda j: (0, j)),
            pl.BlockSpec((_NQ * _D, _B), lambda j: (0, 0)),
        ],
        out_specs=pl.BlockSpec((_V_BLK, _B), lambda j: (j, 0)),
        out_shape=jax.ShapeDtypeStruct((V, _B), jnp.float32),
        compiler_params=pltpu.CompilerParams(
            vmem_limit_bytes=56 * 1024 * 1024),
    )(IL_T, pe4)


def kernel(X, tag, IL, LI):
    V = IL.shape[0]
    main = V - V % 128
    prev_iid = X[:, -2, 3].astype(jnp.int32)
    tail = LI[main:, :]
    pe4 = _make_sc_gather_t(V, _B)(LI.T, tail, prev_iid)
    out_t = _tc_matmul_t(IL.T, pe4)
    return out_t.T


# speedup vs baseline: 2.7487x; 1.0081x over previous
import functools
import math

import jax
import jax.numpy as jnp
from jax import lax
from jax.experimental import pallas as pl
from jax.experimental.pallas import tpu as pltpu
from jax.experimental.pallas import tpu_sc as plsc

_B = 1024
_D = 64
_SCALE = 1.0 / math.sqrt(_D)
_V_BLK = 6144
_SUB = 6272
_NCH = 4
_NQ = 4
_QOWN = 24960


def _make_sc_gather_t(V, B):
    info = plsc.get_sparse_core_info()
    NC, NS, L = info.num_cores, info.num_subcores, info.num_lanes
    NW = NC * NS
    assert NW == 32 and _D == 64 and B % L == 0
    main = V - V % 128
    tail = V - main
    assert _NQ * _QOWN + (_NCH * _SUB - _QOWN) == main
    mesh = plsc.VectorSubcoreMesh(core_axis_name="c", subcore_axis_name="s")

    @functools.partial(
        pl.kernel,
        mesh=mesh,
        out_type=jax.ShapeDtypeStruct((_NQ * _D, B), jnp.float32),
        compiler_params=pltpu.CompilerParams(needs_layout_passes=False),
        scratch_types=[
            pltpu.VMEM((2, 8, _SUB), jnp.float32),
            pltpu.VMEM((tail, _D), jnp.float32),
            pltpu.VMEM((B,), jnp.int32),
            pltpu.VMEM((8, B), jnp.float32),
            pltpu.SemaphoreType.DMA((2,)),
        ],
    )
    def gather_k(lit_hbm, tail_hbm, idx_hbm, out_hbm, buf_v, tail_v, idx_v,
                 acc_v, sems):
        wid = lax.axis_index("s") * NC + lax.axis_index("c")
        gq = wid // _NQ
        q = wid % _NQ
        row0 = pl.multiple_of(8 * gq, 8)
        own_lo = q * _QOWN
        own_hi = jnp.where(q == _NQ - 1, own_lo + _NCH * _SUB,
                           own_lo + _QOWN)

        def chunk_copy(c):
            lo = pl.multiple_of(own_lo + c * _SUB, 128)
            return pltpu.async_copy(
                lit_hbm.at[pl.ds(row0, 8), pl.ds(lo, _SUB)],
                buf_v.at[c % 2], sems.at[c % 2])

        hs = {0: chunk_copy(0)}
        pltpu.sync_copy(idx_hbm, idx_v)
        pltpu.sync_copy(tail_hbm, tail_v)
        for c in range(_NCH):
            if c + 1 < _NCH:
                hs[c + 1] = chunk_copy(c + 1)
            hs[c].wait()
            lo = own_lo + c * _SUB

            def g_body(g, _, c=c, lo=lo):
                iv = idx_v[pl.ds(g * L, L)]
                m = (iv >= lo) & (iv < lo + _SUB) & (iv < own_hi)
                ig = jnp.where(m, iv - lo, 0)
                for rr in range(8):
                    rv = jnp.full((L,), rr, jnp.int32)
                    x = plsc.load_gather(buf_v.at[c % 2], [rv, ig], mask=m)
                    prev = (jnp.zeros_like(x) if c == 0
                            else acc_v[rr, pl.ds(g * L, L)])
                    acc_v[rr, pl.ds(g * L, L)] = jnp.where(m, x, prev)
                return 0

            lax.fori_loop(0, B // L, g_body, 0, unroll=False)

        def t_body(g, _):
            iv = idx_v[pl.ds(g * L, L)]
            m = (iv >= main) & (q == _NQ - 1)
            ig = jnp.where(m, iv - main, 0)
            for rr in range(8):
                rv = jnp.full((L,), rr, jnp.int32)
                x = plsc.load_gather(tail_v, [ig, row0 + rv], mask=m)
                prev = acc_v[rr, pl.ds(g * L, L)]
                acc_v[rr, pl.ds(g * L, L)] = jnp.where(m, x, prev)
            return 0

        lax.fori_loop(0, B // L, t_body, 0, unroll=False)
        pltpu.sync_copy(acc_v, out_hbm.at[pl.ds(pl.multiple_of(q * _D + row0, 8), 8)])

    return gather_k


def _mm_body(il_t_ref, pe4_ref, out_ref):
    pe = (pe4_ref[pl.ds(0, _D), :] + pe4_ref[pl.ds(_D, _D), :]
          + pe4_ref[pl.ds(2 * _D, _D), :] + pe4_ref[pl.ds(3 * _D, _D), :])
    out_ref[...] = lax.dot_general(
        il_t_ref[...], pe,
        dimension_numbers=(((0,), (0,)), ((), ())),
        preferred_element_type=jnp.float32,
    ) * _SCALE


def _tc_matmul_t(IL_T, pe4):
    V = IL_T.shape[1]
    return pl.pallas_call(
        _mm_body,
        grid=(pl.cdiv(V, _V_BLK),),
        in_specs=[
            pl.BlockSpec((_D, _V_BLK), lambda j: (0, j)),
            pl.BlockSpec((_NQ * _D, _B), lambda j: (0, 0)),
        ],
        out_specs=pl.BlockSpec((_V_BLK, _B), lambda j: (j, 0)),
        out_shape=jax.ShapeDtypeStruct((V, _B), jnp.float32),
        compiler_params=pltpu.CompilerParams(
            vmem_limit_bytes=56 * 1024 * 1024),
    )(IL_T, pe4)


def kernel(X, tag, IL, LI):
    V = IL.shape[0]
    main = V - V % 128
    prev_iid = X[:, -2, 3].astype(jnp.int32)
    tail = LI[main:, :]
    pe4 = _make_sc_gather_t(V, _B)(LI.T, tail, prev_iid)
    out_t = _tc_matmul_t(IL.T, pe4)
    return out_t.T
